# baseline (speedup 1.0000x reference)
import numpy as np
import jax
import jax.numpy as jnp
from jax.experimental import pallas as pl
from jax.experimental.pallas import tpu as pltpu

_NF = 64
_NG = 32
_NB = 4
_NS = 6
_B = 4
_LVL = [_NB ** i for i in range(_NS + 1)]
_OFF = [int(v) for v in np.cumsum([0] + _LVL[:-1])]
_NN = sum(_LVL)

_BSZ = [8] + [_B * L for L in _LVL[1:]]
_BOFF = [int(v) for v in np.cumsum([0] + _BSZ[:-1])]
_RTOT = _BOFF[-1] + _BSZ[-1]


def _relu(x):
    return jnp.maximum(x, 0.0)


def _mm(x, W):
    return jnp.dot(x, W, preferred_element_type=jnp.float32)


def _forward_kernel(rv_ref, *refs):
    refs = list(refs)
    a_ref = refs.pop()
    u_ref = refs.pop()
    out_ref = refs.pop()

    def take(n):
        nonlocal refs
        layers = []
        for _ in range(n):
            W = refs.pop(0)[...]
            b = refs.pop(0)[...]
            layers.append((W, b))
        return layers

    gpre = take(2)
    gpost = take(2)
    wproj = take(3)
    wmsg = take(3)
    wupd = take(3)

    rv = rv_ref[...]
    u_ref[0:8, :] = jnp.zeros((8, 128), jnp.float32)
    a_ref[0:8, :] = jnp.zeros((8, 128), jnp.float32)
    u_ref[0:_B, 0:_NF] = rv

    for k in range(1, _NS + 1):
        n_prev = _BOFF[k - 1] + _BSZ[k - 1]
        n_all = _BOFF[k] + _BSZ[k]

        h = _relu(_mm(u_ref[0:n_prev, 0:_NF], gpre[0][0]) + gpre[0][1])
        h = _relu(_mm(h, gpre[1][0]) + gpre[1][1])
        gsum = h[0:_B, :]
        for l in range(1, k):
            L = _LVL[l]
            o = _BOFF[l]
            gsum = gsum + jnp.concatenate(
                [jnp.sum(h[o + b * L:o + (b + 1) * L, :], axis=0,
                         keepdims=True) for b in range(_B)], axis=0)
        g = _relu(_mm(gsum, gpost[0][0]) + gpost[0][1])
        g = _relu(_mm(g, gpost[1][0]) + gpost[1][1])

        for l in range(k):
            if l == 0:
                u_ref[0:_B, _NF:96] = g
            else:
                L = _LVL[l]
                o = _BOFF[l]
                for b in range(_B):
                    u_ref[o + b * L:o + (b + 1) * L, _NF:96] = \
                        jnp.broadcast_to(g[b:b + 1, :], (L, _NG))

        leaf0 = _BOFF[k - 1]
        n_leaf = _B * _LVL[k - 1]
        p = _relu(_mm(u_ref[leaf0:leaf0 + n_leaf, 0:96], wproj[0][0])
                  + wproj[0][1])
        p = _relu(_mm(p, wproj[1][0]) + wproj[1][1])
        p = _relu(_mm(p, wproj[2][0]) + wproj[2][1])
        base = _BOFF[k]
        for c in range(_NB):
            u_ref[pl.Slice(base + c, n_leaf, _NB), 0:_NF] = \
                p[:, c * _NF:(c + 1) * _NF]
        Lk = _LVL[k]
        for b in range(_B):
            u_ref[base + b * Lk:base + (b + 1) * Lk, _NF:96] = \
                jnp.broadcast_to(g[b:b + 1, :], (Lk, _NG))

        m = _relu(_mm(u_ref[0:n_prev, 0:96], wmsg[0][0]) + wmsg[0][1])
        m = _relu(_mm(m, wmsg[1][0]) + wmsg[1][1])
        M = _relu(_mm(m, wmsg[2][0]) + wmsg[2][1])

        a = jnp.zeros((_B, _NF), jnp.float32)
        for l in range(1, k + 1):
            src0 = _BOFF[l - 1]
            n_par = _B * _LVL[l - 1]
            t = a + M[src0:src0 + n_par, :]
            o = _BOFF[l]
            for c in range(_NB):
                a_ref[pl.Slice(o + c, n_par, _NB), 0:_NF] = t
            a = a_ref[o:o + n_par * _NB, 0:_NF]

        u_in = jnp.concatenate(
            [u_ref[0:n_all, 0:_NF], a_ref[0:n_all, 0:_NF],
             u_ref[0:n_all, _NF:96]], axis=1)
        u = _relu(_mm(u_in, wupd[0][0]) + wupd[0][1])
        u = _relu(_mm(u, wupd[1][0]) + wupd[1][1])
        u = _relu(_mm(u, wupd[2][0]) + wupd[2][1])
        if k < _NS:
            u_ref[0:n_all, 0:_NF] = u
        else:
            for b in range(_B):
                for l in range(_NS + 1):
                    L = _LVL[l]
                    o = _BOFF[l] + (b if l == 0 else b * L)
                    out_ref[pl.ds(b * _NN + _OFF[l], L), :] = u[o:o + L, :]


def kernel(random_vector, global_pre, global_post, proj, msg, update):
    flat = []
    for layers in (global_pre, global_post, proj, msg, update):
        for W, b in layers:
            flat.append(W)
            flat.append(b.reshape(1, -1))
    out = pl.pallas_call(
        _forward_kernel,
        out_shape=jax.ShapeDtypeStruct((_B * _NN, _NF), jnp.float32),
        scratch_shapes=[pltpu.VMEM((_RTOT, 128), jnp.float32),
                        pltpu.VMEM((_RTOT, 128), jnp.float32)],
    )(random_vector.reshape(_B, _NF), *flat)
    return out


# speedup vs baseline: 1.0113x; 1.0113x over previous
import numpy as np
import jax
import jax.numpy as jnp
from jax.experimental import pallas as pl
from jax.experimental.pallas import tpu as pltpu

_NF = 64
_NG = 32
_NB = 4
_NS = 6
_B = 4
_LVL = [_NB ** i for i in range(_NS + 1)]
_OFF = [int(v) for v in np.cumsum([0] + _LVL[:-1])]
_NN = sum(_LVL)

_BSZ = [8] + [_B * L for L in _LVL[1:]]
_BOFF = [int(v) for v in np.cumsum([0] + _BSZ[:-1])]
_RTOT = _BOFF[-1] + _BSZ[-1]


def _relu(x):
    return jnp.maximum(x, 0.0)


def _mm(x, W):
    return jnp.dot(x, W, preferred_element_type=jnp.float32)


def _forward_kernel(rv_ref, *refs):
    refs = list(refs)
    a_ref = refs.pop()
    u_ref = refs.pop()
    out_ref = refs.pop()

    def take(n):
        nonlocal refs
        layers = []
        for _ in range(n):
            W = refs.pop(0)[...]
            b = refs.pop(0)[...]
            layers.append((W, b))
        return layers

    gpre = take(2)
    gpost = take(2)
    wproj = take(3)
    wmsg = take(3)
    wupd = take(3)

    rv = rv_ref[...]
    u_ref[0:8, :] = jnp.zeros((8, 128), jnp.float32)
    a_ref[0:8, :] = jnp.zeros((8, 128), jnp.float32)
    u_ref[0:_B, 0:_NF] = rv

    for k in range(1, _NS + 1):
        n_prev = _BOFF[k - 1] + _BSZ[k - 1]
        n_all = _BOFF[k] + _BSZ[k]

        h = _relu(_mm(u_ref[0:n_prev, 0:_NF], gpre[0][0]) + gpre[0][1])
        h = _relu(_mm(h, gpre[1][0]) + gpre[1][1])
        gsum = h[0:_B, :]
        for l in range(1, k):
            L = _LVL[l]
            o = _BOFF[l]
            gsum = gsum + jnp.concatenate(
                [jnp.sum(h[o + b * L:o + (b + 1) * L, :], axis=0,
                         keepdims=True) for b in range(_B)], axis=0)
        g = _relu(_mm(gsum, gpost[0][0]) + gpost[0][1])
        g = _relu(_mm(g, gpost[1][0]) + gpost[1][1])

        for l in range(k):
            if l == 0:
                a_ref[0:_B, _NF:96] = g
            else:
                L = _LVL[l]
                o = _BOFF[l]
                for b in range(_B):
                    a_ref[o + b * L:o + (b + 1) * L, _NF:96] = \
                        jnp.broadcast_to(g[b:b + 1, :], (L, _NG))

        m_in = jnp.concatenate(
            [u_ref[0:n_prev, 0:_NF], a_ref[0:n_prev, _NF:96]], axis=1)
        m = _relu(_mm(m_in, wmsg[0][0]) + wmsg[0][1])
        m = _relu(_mm(m, wmsg[1][0]) + wmsg[1][1])
        M = _relu(_mm(m, wmsg[2][0]) + wmsg[2][1])

        a = jnp.zeros((_B, _NF), jnp.float32)
        for l in range(1, k + 1):
            src0 = _BOFF[l - 1]
            n_par = _B * _LVL[l - 1]
            t = a + M[src0:src0 + n_par, :]
            o = _BOFF[l]
            for c in range(_NB):
                a_ref[pl.Slice(o + c, n_par, _NB), 0:_NF] = t
            a = a_ref[o:o + n_par * _NB, 0:_NF]

        for l in range(1, k + 1):
            L = _LVL[l]
            o = _BOFF[l]
            for b in range(_B):
                a_ref[o + b * L:o + (b + 1) * L, _NF:96] = \
                    jnp.broadcast_to(g[b:b + 1, :], (L, _NG))

        leaf0 = _BOFF[k - 1]
        n_leaf = _B * _LVL[k - 1]
        p_in = jnp.concatenate(
            [u_ref[leaf0:leaf0 + n_leaf, 0:_NF],
             a_ref[leaf0:leaf0 + n_leaf, _NF:96]], axis=1)
        p = _relu(_mm(p_in, wproj[0][0]) + wproj[0][1])
        p = _relu(_mm(p, wproj[1][0]) + wproj[1][1])
        p = _relu(_mm(p, wproj[2][0]) + wproj[2][1])
        base = _BOFF[k]
        for c in range(_NB):
            u_ref[pl.Slice(base + c, n_leaf, _NB), 0:_NF] = \
                p[:, c * _NF:(c + 1) * _NF]

        u_in = jnp.concatenate(
            [u_ref[0:n_all, 0:_NF], a_ref[0:n_all, 0:96]], axis=1)
        u = _relu(_mm(u_in, wupd[0][0]) + wupd[0][1])
        u = _relu(_mm(u, wupd[1][0]) + wupd[1][1])
        u = _relu(_mm(u, wupd[2][0]) + wupd[2][1])
        if k < _NS:
            u_ref[0:n_all, 0:_NF] = u
        else:
            for b in range(_B):
                for l in range(_NS + 1):
                    L = _LVL[l]
                    o = _BOFF[l] + (b if l == 0 else b * L)
                    out_ref[pl.ds(b * _NN + _OFF[l], L), :] = u[o:o + L, :]


def kernel(random_vector, global_pre, global_post, proj, msg, update):
    flat = []
    for layers in (global_pre, global_post, proj, msg, update):
        for W, b in layers:
            flat.append(W)
            flat.append(b.reshape(1, -1))
    out = pl.pallas_call(
        _forward_kernel,
        out_shape=jax.ShapeDtypeStruct((_B * _NN, _NF), jnp.float32),
        scratch_shapes=[pltpu.VMEM((_RTOT, 128), jnp.float32),
                        pltpu.VMEM((_RTOT, 128), jnp.float32)],
    )(random_vector.reshape(_B, _NF), *flat)
    return out
